# baseline (speedup 1.0000x reference)
import functools

import jax
import jax.numpy as jnp
from jax import lax
from jax.experimental import pallas as pl
from jax.experimental.pallas import tpu as pltpu
from jax.experimental.pallas import tpu_sc as plsc

B = 16384
F = 26
V = 100000
D = 16

NC = 2
NS = 16
NW = NC * NS
SPT = B // NW
C = 128
NCHUNK = SPT // C


def _fm_body(idxT_hbm, emb_hbm, lin_hbm, linb_hbm, out_hbm,
             idx_v, vals_v, lin_v, out_v, linb_v, sem0, sem1):
    cid = lax.axis_index("c")
    sid = lax.axis_index("s")
    wid = sid * NC + cid

    pltpu.sync_copy(linb_hbm, linb_v)
    bvec = linb_v[...]
    sems = (sem0, sem1)

    def fire(c):
        buf = c % 2
        base = wid * SPT + c * C
        sem = sems[buf]

        def idx_body(f, carry):
            pltpu.sync_copy(idxT_hbm.at[f, pl.ds(base, C)], idx_v.at[buf, f])
            return carry

        lax.fori_loop(0, F, idx_body, 0)

        def emb_body(fd, carry):
            f = fd // D
            pltpu.async_copy(
                emb_hbm.at[pl.ds(fd * V, V)].at[idx_v.at[buf, f]],
                vals_v.at[buf, fd], sem)
            return carry

        lax.fori_loop(0, F * D, emb_body, 0)

        def lin_body(f, carry):
            pltpu.async_copy(
                lin_hbm.at[pl.ds(f * V, V)].at[idx_v.at[buf, f]],
                lin_v.at[buf, f], sem)
            return carry

        lax.fori_loop(0, F, lin_body, 0)

    def drain(c):
        buf = c % 2
        sem = sems[buf]

        def emb_body(fd, carry):
            f = fd // D
            pltpu.make_async_copy(
                emb_hbm.at[pl.ds(fd * V, V)].at[idx_v.at[buf, f]],
                vals_v.at[buf, fd], sem).wait()
            return carry

        lax.fori_loop(0, F * D, emb_body, 0)

        def lin_body(f, carry):
            pltpu.make_async_copy(
                lin_hbm.at[pl.ds(f * V, V)].at[idx_v.at[buf, f]],
                lin_v.at[buf, f], sem).wait()
            return carry

        lax.fori_loop(0, F, lin_body, 0)

    def compute(c):
        buf = c % 2
        base = wid * SPT + c * C

        def group_body(g, carry):
            sl = pl.ds(g * 16, 16)
            acc = bvec
            lt = lin_v[buf, 0, sl]
            for f in range(1, F):
                lt = lt + lin_v[buf, f, sl]
            acc = acc + lt
            for d in range(D):
                s = vals_v[buf, d, sl]
                ss = s * s
                for f in range(1, F):
                    v = vals_v[buf, f * D + d, sl]
                    s = s + v
                    ss = ss + v * v
                acc = acc + 0.5 * (s * s - ss)
            out_v[sl] = 1.0 / (1.0 + jnp.exp(-acc))
            return carry

        lax.fori_loop(0, C // 16, group_body, 0)
        pltpu.sync_copy(out_v, out_hbm.at[pl.ds(base, C)])

    fire(0)
    for c in range(NCHUNK):
        if c + 1 < NCHUNK:
            fire(c + 1)
        drain(c)
        compute(c)


@functools.cache
def _fm_sc():
    return pl.kernel(
        _fm_body,
        out_type=jax.ShapeDtypeStruct((B,), jnp.float32),
        mesh=plsc.VectorSubcoreMesh(
            core_axis_name="c", subcore_axis_name="s",
            num_cores=NC, num_subcores=NS),
        compiler_params=pltpu.CompilerParams(
            needs_layout_passes=False, use_tc_tiling_on_sc=False),
        scratch_types=[
            pltpu.VMEM((2, F, C), jnp.int32),
            pltpu.VMEM((2, F * D, C), jnp.float32),
            pltpu.VMEM((2, F, C), jnp.float32),
            pltpu.VMEM((C,), jnp.float32),
            pltpu.VMEM((16,), jnp.float32),
            pltpu.SemaphoreType.DMA,
            pltpu.SemaphoreType.DMA,
        ],
    )


def kernel(indices, emb_table, lin_w, lin_b):
    idxT = indices.T.astype(jnp.int32)
    emb_flat = jnp.transpose(emb_table, (0, 2, 1)).reshape(F * D * V)
    lin_flat = lin_w.reshape(F * V)
    linb16 = jnp.tile(lin_b.astype(jnp.float32), 16)
    out = _fm_sc()(idxT, emb_flat, lin_flat, linb16)
    return out.reshape(B, 1)


# speedup vs baseline: 1.3004x; 1.3004x over previous
import functools

import jax
import jax.numpy as jnp
from jax import lax
from jax.experimental import pallas as pl
from jax.experimental.pallas import tpu as pltpu
from jax.experimental.pallas import tpu_sc as plsc


def _detile_body(in_ref, out_ref):
    for d in range(8):
        out_ref[pl.ds(d * 100352, 100000)] = in_ref[0, d, :]


def _detile(emb_t):
    return pl.pallas_call(
        _detile_body,
        grid=(26, 2),
        in_specs=[pl.BlockSpec((1, 8, 100000), lambda f, t: (f, t, 0))],
        out_specs=pl.BlockSpec((8 * 100352,), lambda f, t: (f * 2 + t,)),
        out_shape=jax.ShapeDtypeStruct((26 * 16 * 100352,), jnp.float32),
    )(emb_t)

B = 16384
F = 26
V = 100000
D = 16
VP = 100352

NC = 2
NS = 16
NW = NC * NS
SPT = B // NW
C = 256
NCHUNK = SPT // C


def _fm_body(idxT_hbm, emb_hbm, lin_hbm, linb_hbm, out_hbm,
             idx_v, vals_v, lin_v, out_v, linb_v, sem):
    cid = lax.axis_index("c")
    sid = lax.axis_index("s")
    wid = sid * NC + cid

    pltpu.sync_copy(linb_hbm, linb_v)
    bvec = linb_v[...]

    def chunk_body(c, carry):
        base = wid * SPT + c * C
        for f in range(F):
            pltpu.sync_copy(idxT_hbm.at[f, pl.ds(base, C)], idx_v.at[f])
        copies = []
        for f in range(F):
            for d in range(D):
                copies.append(pltpu.async_copy(
                    emb_hbm.at[pl.ds((f * D + d) * VP, V)].at[idx_v.at[f]],
                    vals_v.at[f * D + d], sem))
            copies.append(pltpu.async_copy(
                lin_hbm.at[pl.ds(f * V, V)].at[idx_v.at[f]], lin_v.at[f],
                sem))
        for cp in copies:
            cp.wait()

        def group_body(g, carry2):
            sl = pl.ds(g * 16, 16)
            acc = bvec
            lt = lin_v[0, sl]
            for f in range(1, F):
                lt = lt + lin_v[f, sl]
            acc = acc + lt
            for d in range(D):
                s = vals_v[d, sl]
                ss = s * s
                for f in range(1, F):
                    v = vals_v[f * D + d, sl]
                    s = s + v
                    ss = ss + v * v
                acc = acc + 0.5 * (s * s - ss)
            out_v[sl] = 1.0 / (1.0 + jnp.exp(-acc))
            return carry2

        lax.fori_loop(0, C // 16, group_body, 0)
        pltpu.sync_copy(out_v, out_hbm.at[pl.ds(base, C)])
        return carry

    lax.fori_loop(0, NCHUNK, chunk_body, 0)


@functools.cache
def _fm_sc():
    return pl.kernel(
        _fm_body,
        out_type=jax.ShapeDtypeStruct((B,), jnp.float32),
        mesh=plsc.VectorSubcoreMesh(
            core_axis_name="c", subcore_axis_name="s",
            num_cores=NC, num_subcores=NS),
        compiler_params=pltpu.CompilerParams(
            needs_layout_passes=False, use_tc_tiling_on_sc=False),
        scratch_types=[
            pltpu.VMEM((F, C), jnp.int32),
            pltpu.VMEM((F * D, C), jnp.float32),
            pltpu.VMEM((F, C), jnp.float32),
            pltpu.VMEM((C,), jnp.float32),
            pltpu.VMEM((16,), jnp.float32),
            pltpu.SemaphoreType.DMA,
        ],
    )


def kernel(indices, emb_table, lin_w, lin_b):
    idxT = indices.T.astype(jnp.int32)
    emb_flat = _detile(jnp.transpose(emb_table, (0, 2, 1)))
    lin_flat = lin_w.reshape(F * V)
    linb16 = jnp.tile(lin_b.astype(jnp.float32), 16)
    out = _fm_sc()(idxT, emb_flat, lin_flat, linb16)
    return out.reshape(B, 1)
